# Initial kernel scaffold; baseline (speedup 1.0000x reference)
import jax
import jax.numpy as jnp
from jax.experimental import pallas as pl


def kernel(feature, idx):
    raise NotImplementedError("write your pallas kernel here")


# speedup vs baseline: 536.2282x; 536.2282x over previous
import jax
import jax.numpy as jnp
from jax import lax
from jax.experimental import pallas as pl
from jax.experimental.pallas import tpu as pltpu
from jax.experimental.pallas import tpu_sc as plsc

_B, _C, _N = 8, 64, 16384
_PS = 1024 * 32

_NC, _NS = 2, 16
_NW = _NC * _NS
_TPB = _NW // _B
_CPW = _C // _TPB
_L = 16


def _body(feat_hbm, idx_hbm, out_hbm, idx_v, row_v, out_v):
    cid = lax.axis_index("c")
    sid = lax.axis_index("s")
    wid = sid * _NC + cid
    b = wid // _TPB
    cg = wid % _TPB
    pltpu.sync_copy(idx_hbm.at[b], idx_v)

    @pl.loop(0, _CPW)
    def _chan(cl):
        row = b * _C + cg * _CPW + cl
        pltpu.sync_copy(feat_hbm.at[row], row_v)

        @pl.loop(0, _PS // _L, unroll=8)
        def _vec(i):
            iv = idx_v[pl.ds(i * _L, _L)]
            out_v[pl.ds(i * _L, _L)] = plsc.load_gather(row_v, [iv])

        pltpu.sync_copy(out_v, out_hbm.at[row])


def kernel(feature, idx):
    B, C, N = feature.shape
    _, P, S = idx.shape
    feat2 = feature.reshape(B * C, N)
    idx2 = idx.reshape(B, P * S)
    mesh = plsc.VectorSubcoreMesh(
        core_axis_name="c", subcore_axis_name="s", num_cores=_NC, num_subcores=_NS
    )
    f = pl.kernel(
        _body,
        out_type=jax.ShapeDtypeStruct((B * C, P * S), jnp.float32),
        mesh=mesh,
        scratch_types=[
            pltpu.VMEM((_PS,), jnp.int32),
            pltpu.VMEM((_N,), jnp.float32),
            pltpu.VMEM((_PS,), jnp.float32),
        ],
        compiler_params=pltpu.CompilerParams(needs_layout_passes=False),
    )
    out = f(feat2, idx2)
    return out.reshape(B, C, P, S)
